# speedup vs baseline: 1.0406x; 1.0406x over previous
import functools

import numpy as np
import jax
import jax.numpy as jnp
from jax import lax
from jax.experimental import pallas as pl
from jax.experimental.pallas import tpu as pltpu
from jax.experimental.pallas import tpu_sc as plsc

_VOCAB = 100000
_D = 512
_B = 4
_S = 4096

_NC = 2
_NS = 16
_NW = _NC * _NS
_SPW = _S // _NW
_C = 64
_HALVES = _SPW // _C


def _pe_table() -> np.ndarray:
    pos = np.arange(_S, dtype=np.float32)[:, None]
    div = np.exp(np.arange(0, _D, 2, dtype=np.float32) * (-np.log(10000.0) / _D))
    pe = np.zeros((_S, _D), np.float32)
    pe[:, 0::2] = np.sin(pos * div)
    pe[:, 1::2] = np.cos(pos * div)
    return pe


_PE = _pe_table()


@functools.partial(
    pl.kernel,
    out_type=jax.ShapeDtypeStruct((_B, _S, _D), jnp.float32),
    mesh=plsc.VectorSubcoreMesh(core_axis_name="c", subcore_axis_name="s"),
    scratch_types=[
        pltpu.VMEM((_C,), jnp.int32),
        pltpu.VMEM((_C, _D), jnp.float32),
        pltpu.VMEM((_C, _D), jnp.float32),
        pltpu.SemaphoreType.DMA,
    ],
)
def _embed(x_hbm, pe_hbm, table_hbm, out_hbm, idx_v, pe_v, row_v, sem):
    wid = lax.axis_index("s") * _NC + lax.axis_index("c")
    s_base = wid * _SPW
    for h in range(_HALVES):
        off = s_base + h * _C
        pltpu.sync_copy(pe_hbm.at[pl.ds(off, _C)], pe_v)
        for b in range(_B):
            pltpu.sync_copy(x_hbm.at[b, pl.ds(off, _C)], idx_v)
            pltpu.async_copy(table_hbm.at[idx_v], row_v, sem).wait()

            def _row(i, _):
                def _vec(j, _):
                    sl = pl.ds(j * 16, 16)
                    row_v[i, sl] = row_v[i, sl] + pe_v[i, sl]
                    return 0
                return lax.fori_loop(0, _D // 16, _vec, 0)

            lax.fori_loop(0, _C, _row, 0)
            pltpu.sync_copy(row_v, out_hbm.at[b, pl.ds(off, _C)])


def kernel(x, table):
    x = x.astype(jnp.int32)
    pe = jnp.asarray(_PE)
    return _embed(x, pe, table)


# speedup vs baseline: 2.1913x; 2.1058x over previous
import functools

import numpy as np
import jax
import jax.numpy as jnp
from jax import lax
from jax.experimental import pallas as pl
from jax.experimental.pallas import tpu as pltpu
from jax.experimental.pallas import tpu_sc as plsc

_VOCAB = 100000
_D = 512
_B = 4
_S = 4096

_NC = 2
_NS = 16
_NW = _NC * _NS
_SPW = _S // _NW
_C = 64
_NCHUNK = (_SPW // _C) * _B


def _pe_table() -> np.ndarray:
    pos = np.arange(_S, dtype=np.float32)[:, None]
    div = np.exp(np.arange(0, _D, 2, dtype=np.float32) * (-np.log(10000.0) / _D))
    pe = np.zeros((_S, _D), np.float32)
    pe[:, 0::2] = np.sin(pos * div)
    pe[:, 1::2] = np.cos(pos * div)
    return pe


_PE = _pe_table()


@functools.partial(
    pl.kernel,
    out_type=jax.ShapeDtypeStruct((_B, _S, _D), jnp.float32),
    mesh=plsc.VectorSubcoreMesh(core_axis_name="c", subcore_axis_name="s"),
    scratch_types=[
        pltpu.VMEM((2, _C), jnp.int32),
        pltpu.VMEM((_C, _D), jnp.float32),
        pltpu.VMEM((2, _C, _D), jnp.float32),
        pltpu.SemaphoreType.DMA,
        pltpu.SemaphoreType.DMA,
        pltpu.SemaphoreType.DMA,
        pltpu.SemaphoreType.DMA,
    ],
)
def _embed(x_hbm, pe_hbm, table_hbm, out_hbm, idx_v, pe_v, row_v,
           gsem0, gsem1, osem0, osem1):
    wid = lax.axis_index("s") * _NC + lax.axis_index("c")
    s_base = wid * _SPW
    gsem = (gsem0, gsem1)
    osem = (osem0, osem1)

    def chunk_bh(t):
        h, b = divmod(t, _B)
        return b, s_base + h * _C

    def start_gather(t):
        slot = t % 2
        b, off = chunk_bh(t)
        pltpu.sync_copy(x_hbm.at[b, pl.ds(off, _C)], idx_v.at[slot])
        return pltpu.async_copy(table_hbm.at[idx_v.at[slot]],
                                row_v.at[slot], gsem[slot])

    pltpu.sync_copy(pe_hbm.at[pl.ds(s_base, _C)], pe_v)
    gather = start_gather(0)
    pending_out = [None, None]
    for t in range(_NCHUNK):
        slot = t % 2
        b, off = chunk_bh(t)
        next_gather = None
        if t + 1 < _NCHUNK:
            if pending_out[1 - slot] is not None:
                pending_out[1 - slot].wait()
            next_gather = start_gather(t + 1)
        gather.wait()

        def _row(i, _):
            for j in range(_D // 16):
                sl = pl.ds(j * 16, 16)
                plsc.addupdate(row_v.at[slot, i, sl], pe_v[i, sl])
            return 0

        lax.fori_loop(0, _C, _row, 0)
        pending_out[slot] = pltpu.async_copy(
            row_v.at[slot], out_hbm.at[b, pl.ds(off, _C)], osem[slot])
        if t == _B - 1 and _NCHUNK > _B:
            pltpu.sync_copy(pe_hbm.at[pl.ds(s_base + _C, _C)], pe_v)
        gather = next_gather
    for p in pending_out:
        if p is not None:
            p.wait()


def kernel(x, table):
    x = x.astype(jnp.int32)
    pe = jnp.asarray(_PE)
    return _embed(x, pe, table)
